# baseline (speedup 1.0000x reference)
import functools

import jax
import jax.numpy as jnp
from jax import lax
from jax.experimental import pallas as pl
from jax.experimental.pallas import tpu as pltpu
from jax.experimental.pallas import tpu_sc as plsc

BATCH = 1024
LATENT = 256
K_TOTAL = 100000
BK = 2816
NFULL = K_TOTAL // BK
TAIL_W = K_TOTAL - NFULL * BK
TAIL_A = TAIL_W + 1

NBUF = 3
NSA = 8
NSW = 2
ABAND = BK // NSA
WBAND = BK // NSW


def _mm_body(xt_hbm, wt_hbm, b_ref, o_ref,
             a_bufs, w_bufs, a_tail, w_tail, carry_ref,
             a_sems, w_sems, t_sems):
    k = pl.program_id(0)
    slot = jax.lax.rem(k, NBUF)

    def start_full(i, s):
        for q in range(NSA):
            pltpu.make_async_copy(
                xt_hbm.at[pl.ds(i * BK + q * ABAND, ABAND), :],
                a_bufs.at[s, pl.ds(q * ABAND, ABAND)], a_sems.at[s, q],
            ).start(priority=q % 2)
        for q in range(NSW):
            pltpu.make_async_copy(
                wt_hbm.at[pl.ds(i * BK + q * WBAND, WBAND), :],
                w_bufs.at[s, pl.ds(q * WBAND, WBAND)], w_sems.at[s, q],
            ).start(priority=q % 2)

    def wait_full(s):
        for q in range(NSA):
            pltpu.make_async_copy(
                xt_hbm.at[pl.ds(0, ABAND), :],
                a_bufs.at[s, pl.ds(0, ABAND)], a_sems.at[s, q],
            ).wait()
        for q in range(NSW):
            pltpu.make_async_copy(
                wt_hbm.at[pl.ds(0, WBAND), :],
                w_bufs.at[s, pl.ds(0, WBAND)], w_sems.at[s, q],
            ).wait()

    @pl.when(k == 0)
    def _():
        carry_ref[...] = jnp.zeros((1, LATENT), jnp.float32)
        for i in range(NBUF - 1):
            start_full(i, i)

    @pl.when(k + NBUF - 1 < NFULL)
    def _():
        start_full(k + NBUF - 1, jax.lax.rem(k + NBUF - 1, NBUF))

    @pl.when(k + NBUF - 1 == NFULL)
    def _():
        pltpu.make_async_copy(
            xt_hbm.at[pl.ds(NFULL * BK, TAIL_A), :], a_tail, t_sems.at[0]
        ).start(priority=0)
        pltpu.make_async_copy(
            wt_hbm.at[pl.ds(NFULL * BK, TAIL_W), :], w_tail, t_sems.at[1]
        ).start(priority=1)

    carry_row = carry_ref[...]

    @pl.when(k < NFULL)
    def _():
        wait_full(slot)
        wk = w_bufs[slot]
        rolled = pltpu.roll(wk, 1, 0)
        sub = lax.broadcasted_iota(jnp.int32, (BK, LATENT), 0)
        wshift = jnp.where(sub == 0, carry_row, rolled)
        carry_ref[...] = wk[BK - 1:BK, :]
        acc = lax.dot_general(
            a_bufs[slot].astype(jnp.bfloat16), wshift.astype(jnp.bfloat16),
            (((0,), (0,)), ((), ())),
            preferred_element_type=jnp.float32,
        )

        @pl.when(k == 0)
        def _():
            o_ref[...] = acc + b_ref[...]

        @pl.when(k > 0)
        def _():
            o_ref[...] += acc

    @pl.when(k == NFULL)
    def _():
        pltpu.make_async_copy(
            xt_hbm.at[pl.ds(NFULL * BK, TAIL_A), :], a_tail, t_sems.at[0]
        ).wait()
        pltpu.make_async_copy(
            wt_hbm.at[pl.ds(NFULL * BK, TAIL_W), :], w_tail, t_sems.at[1]
        ).wait()
        wsh = jnp.concatenate(
            [carry_row, w_tail[...]], axis=0)
        o_ref[...] += lax.dot_general(
            a_tail[...].astype(jnp.bfloat16), wsh.astype(jnp.bfloat16),
            (((0,), (0,)), ((), ())),
            preferred_element_type=jnp.float32,
        )


def _matmul(xt, wt, b2d):
    return pl.pallas_call(
        _mm_body,
        grid=(NFULL + 1,),
        in_specs=[
            pl.BlockSpec(memory_space=pltpu.MemorySpace.HBM),
            pl.BlockSpec(memory_space=pltpu.MemorySpace.HBM),
            pl.BlockSpec((1, LATENT), lambda k: (0, 0)),
        ],
        out_specs=pl.BlockSpec((BATCH, LATENT), lambda k: (0, 0)),
        out_shape=jax.ShapeDtypeStruct((BATCH, LATENT), jnp.float32),
        scratch_shapes=[
            pltpu.VMEM((NBUF, BK, BATCH), jnp.float32),
            pltpu.VMEM((NBUF, BK, LATENT), jnp.float32),
            pltpu.VMEM((TAIL_A, BATCH), jnp.float32),
            pltpu.VMEM((TAIL_W, LATENT), jnp.float32),
            pltpu.VMEM((1, LATENT), jnp.float32),
            pltpu.SemaphoreType.DMA((NBUF, NSA)),
            pltpu.SemaphoreType.DMA((NBUF, NSW)),
            pltpu.SemaphoreType.DMA((2,)),
        ],
        compiler_params=pltpu.CompilerParams(
            dimension_semantics=("arbitrary",),
        ),
    )(xt, wt, b2d)


def _make_sc_gather():
    info = plsc.get_sparse_core_info()
    nc, ns = info.num_cores, info.num_subcores
    nw = nc * ns
    b_per_w = BATCH // nw
    mesh = plsc.VectorSubcoreMesh(core_axis_name="c", subcore_axis_name="s")

    @functools.partial(
        pl.kernel,
        mesh=mesh,
        out_type=jax.ShapeDtypeStruct((BATCH, LATENT), jnp.float32),
        scratch_types=[
            pltpu.VMEM((b_per_w,), jnp.int32),
            pltpu.VMEM((b_per_w, LATENT), jnp.float32),
            pltpu.SemaphoreType.DMA,
        ],
    )
    def gather(table_hbm, idx_hbm, out_hbm, idx_v, rows_v, sem):
        wid = lax.axis_index("s") * nc + lax.axis_index("c")
        base = wid * b_per_w
        pltpu.sync_copy(idx_hbm.at[pl.ds(base, b_per_w)], idx_v)
        pltpu.async_copy(table_hbm.at[idx_v], rows_v, sem).wait()
        pltpu.sync_copy(rows_v, out_hbm.at[pl.ds(base, b_per_w)])

    return gather


_sc_gather = None


def kernel(input_data, users_embedding, W, b):
    global _sc_gather
    if _sc_gather is None:
        _sc_gather = _make_sc_gather()
    user_ids = input_data[:, 0].astype(jnp.int32)
    users_embed = _sc_gather(users_embedding, user_ids)
    mm = _matmul(input_data.T, W.T, b.reshape(1, LATENT))
    return mm + users_embed


# speedup vs baseline: 1.0148x; 1.0148x over previous
import functools

import jax
import jax.numpy as jnp
from jax import lax
from jax.experimental import pallas as pl
from jax.experimental.pallas import tpu as pltpu
from jax.experimental.pallas import tpu_sc as plsc

BATCH = 1024
LATENT = 256
K_TOTAL = 100000
BK = 1408
NFULL = K_TOTAL // BK
TAIL_W = K_TOTAL - NFULL * BK
TAIL_A = TAIL_W + 1

NBUF = 6
NSA = 4
NSW = 2
ABAND = BK // NSA
WBAND = BK // NSW


def _mm_body(xt_hbm, wt_hbm, b_ref, o_ref,
             a_bufs, w_bufs, a_tail, w_tail, carry_ref,
             a_sems, w_sems, t_sems):
    k = pl.program_id(0)
    slot = jax.lax.rem(k, NBUF)

    def start_full(i, s):
        for q in range(NSA):
            pltpu.make_async_copy(
                xt_hbm.at[pl.ds(i * BK + q * ABAND, ABAND), :],
                a_bufs.at[s, pl.ds(q * ABAND, ABAND)], a_sems.at[s, q],
            ).start(priority=q % 2)
        for q in range(NSW):
            pltpu.make_async_copy(
                wt_hbm.at[pl.ds(i * BK + q * WBAND, WBAND), :],
                w_bufs.at[s, pl.ds(q * WBAND, WBAND)], w_sems.at[s, q],
            ).start(priority=q % 2)

    def wait_full(s):
        for q in range(NSA):
            pltpu.make_async_copy(
                xt_hbm.at[pl.ds(0, ABAND), :],
                a_bufs.at[s, pl.ds(0, ABAND)], a_sems.at[s, q],
            ).wait()
        for q in range(NSW):
            pltpu.make_async_copy(
                wt_hbm.at[pl.ds(0, WBAND), :],
                w_bufs.at[s, pl.ds(0, WBAND)], w_sems.at[s, q],
            ).wait()

    @pl.when(k == 0)
    def _():
        carry_ref[...] = jnp.zeros((1, LATENT), jnp.float32)
        for i in range(NBUF - 1):
            start_full(i, i)

    @pl.when(k + NBUF - 1 < NFULL)
    def _():
        start_full(k + NBUF - 1, jax.lax.rem(k + NBUF - 1, NBUF))

    @pl.when(k + NBUF - 1 == NFULL)
    def _():
        pltpu.make_async_copy(
            xt_hbm.at[pl.ds(NFULL * BK, TAIL_A), :], a_tail, t_sems.at[0]
        ).start(priority=0)
        pltpu.make_async_copy(
            wt_hbm.at[pl.ds(NFULL * BK, TAIL_W), :], w_tail, t_sems.at[1]
        ).start(priority=1)

    carry_row = carry_ref[...]

    @pl.when(k < NFULL)
    def _():
        wait_full(slot)
        wk = w_bufs[slot]
        rolled = pltpu.roll(wk, 1, 0)
        sub = lax.broadcasted_iota(jnp.int32, (BK, LATENT), 0)
        wshift = jnp.where(sub == 0, carry_row, rolled)
        carry_ref[...] = wk[BK - 1:BK, :]
        acc = lax.dot_general(
            a_bufs[slot].astype(jnp.bfloat16), wshift.astype(jnp.bfloat16),
            (((0,), (0,)), ((), ())),
            preferred_element_type=jnp.float32,
        )

        @pl.when(k == 0)
        def _():
            o_ref[...] = acc + b_ref[...]

        @pl.when(k > 0)
        def _():
            o_ref[...] += acc

    @pl.when(k == NFULL)
    def _():
        pltpu.make_async_copy(
            xt_hbm.at[pl.ds(NFULL * BK, TAIL_A), :], a_tail, t_sems.at[0]
        ).wait()
        pltpu.make_async_copy(
            wt_hbm.at[pl.ds(NFULL * BK, TAIL_W), :], w_tail, t_sems.at[1]
        ).wait()
        wsh = jnp.concatenate(
            [carry_row, w_tail[...]], axis=0)
        o_ref[...] += lax.dot_general(
            a_tail[...].astype(jnp.bfloat16), wsh.astype(jnp.bfloat16),
            (((0,), (0,)), ((), ())),
            preferred_element_type=jnp.float32,
        )


def _matmul(xt, wt, b2d):
    return pl.pallas_call(
        _mm_body,
        grid=(NFULL + 1,),
        in_specs=[
            pl.BlockSpec(memory_space=pltpu.MemorySpace.HBM),
            pl.BlockSpec(memory_space=pltpu.MemorySpace.HBM),
            pl.BlockSpec((1, LATENT), lambda k: (0, 0)),
        ],
        out_specs=pl.BlockSpec((BATCH, LATENT), lambda k: (0, 0)),
        out_shape=jax.ShapeDtypeStruct((BATCH, LATENT), jnp.float32),
        scratch_shapes=[
            pltpu.VMEM((NBUF, BK, BATCH), jnp.float32),
            pltpu.VMEM((NBUF, BK, LATENT), jnp.float32),
            pltpu.VMEM((TAIL_A, BATCH), jnp.float32),
            pltpu.VMEM((TAIL_W, LATENT), jnp.float32),
            pltpu.VMEM((1, LATENT), jnp.float32),
            pltpu.SemaphoreType.DMA((NBUF, NSA)),
            pltpu.SemaphoreType.DMA((NBUF, NSW)),
            pltpu.SemaphoreType.DMA((2,)),
        ],
        compiler_params=pltpu.CompilerParams(
            dimension_semantics=("arbitrary",),
        ),
    )(xt, wt, b2d)


def _make_sc_gather():
    info = plsc.get_sparse_core_info()
    nc, ns = info.num_cores, info.num_subcores
    nw = nc * ns
    b_per_w = BATCH // nw
    mesh = plsc.VectorSubcoreMesh(core_axis_name="c", subcore_axis_name="s")

    @functools.partial(
        pl.kernel,
        mesh=mesh,
        out_type=jax.ShapeDtypeStruct((BATCH, LATENT), jnp.float32),
        scratch_types=[
            pltpu.VMEM((b_per_w,), jnp.int32),
            pltpu.VMEM((b_per_w, LATENT), jnp.float32),
            pltpu.SemaphoreType.DMA,
        ],
    )
    def gather(table_hbm, idx_hbm, out_hbm, idx_v, rows_v, sem):
        wid = lax.axis_index("s") * nc + lax.axis_index("c")
        base = wid * b_per_w
        pltpu.sync_copy(idx_hbm.at[pl.ds(base, b_per_w)], idx_v)
        pltpu.async_copy(table_hbm.at[idx_v], rows_v, sem).wait()
        pltpu.sync_copy(rows_v, out_hbm.at[pl.ds(base, b_per_w)])

    return gather


_sc_gather = None


def kernel(input_data, users_embedding, W, b):
    global _sc_gather
    if _sc_gather is None:
        _sc_gather = _make_sc_gather()
    user_ids = input_data[:, 0].astype(jnp.int32)
    users_embed = _sc_gather(users_embedding, user_ids)
    mm = _matmul(input_data.T, W.T, b.reshape(1, LATENT))
    return mm + users_embed
